# baseline (speedup 1.0000x reference)
import jax
import jax.numpy as jnp
from jax.experimental import pallas as pl
from jax.experimental.pallas import tpu as pltpu


def _dma_copy(x_ref, o_ref, sem):
    copy = pltpu.make_async_copy(x_ref, o_ref, sem)
    copy.start()
    copy.wait()


def kernel(X):
    return pl.pallas_call(
        _dma_copy,
        in_specs=[pl.BlockSpec(memory_space=pl.ANY)],
        out_specs=pl.BlockSpec(memory_space=pl.ANY),
        out_shape=jax.ShapeDtypeStruct(X.shape, X.dtype),
        scratch_shapes=[pltpu.SemaphoreType.DMA],
    )(X)


# speedup vs baseline: 10.0506x; 10.0506x over previous
import jax
import jax.numpy as jnp
from jax.experimental import pallas as pl
from jax.experimental.pallas import tpu as pltpu


def _copy_block(x_ref, o_ref):
    o_ref[...] = x_ref[...]


def kernel(X):
    b, n, f = X.shape
    flat = X.reshape(b * n, f)
    rows = b * n
    grid = (4,)
    blk = rows // 4
    out = pl.pallas_call(
        _copy_block,
        grid=grid,
        in_specs=[pl.BlockSpec((blk, f), lambda i: (i, 0))],
        out_specs=pl.BlockSpec((blk, f), lambda i: (i, 0)),
        out_shape=jax.ShapeDtypeStruct((rows, f), X.dtype),
    )(flat)
    return out.reshape(b, n, f)
